# baseline (speedup 1.0000x reference)
import functools

import jax
import jax.numpy as jnp
from jax import lax
from jax.experimental import pallas as pl
from jax.experimental.pallas import tpu as pltpu
from jax.experimental.pallas import tpu_sc as plsc

N_NODES = 10000
N_PAD = 10240
N_EDGES = 320000
E_PAD = 327680
D = 128
NC = 2
NS = 16
NW = NC * NS
E_PER_W = E_PAD // NW
CHUNK = 80
NCHUNK = E_PER_W // CHUNK
ROWS_PER_TILE = N_PAD // NS
RBLK = 1024

_sc_mesh = plsc.VectorSubcoreMesh(core_axis_name="c", subcore_axis_name="s")


@functools.partial(
    pl.kernel,
    out_type=jax.ShapeDtypeStruct((NC, N_PAD, 16), jnp.float32),
    mesh=_sc_mesh,
    scratch_types=[
        pltpu.VMEM((NCHUNK, CHUNK), jnp.int32),
        pltpu.VMEM((CHUNK, 16), jnp.float32),
        pltpu.VMEM_SHARED((N_PAD, 16), jnp.float32),
    ],
)
def _sc_deg(dst_hbm, ones_hbm, out_hbm, dst_v, ones_v, deg):
    c = lax.axis_index("c")
    s = lax.axis_index("s")
    wid = c * NS + s
    row0 = s * ROWS_PER_TILE
    pltpu.sync_copy(ones_hbm, deg.at[pl.ds(row0, ROWS_PER_TILE)])
    pltpu.sync_copy(ones_hbm.at[pl.ds(0, CHUNK)], ones_v)
    pltpu.sync_copy(dst_hbm.at[wid], dst_v)
    plsc.subcore_barrier()

    def body(j, carry):
        pltpu.sync_copy(ones_v, deg.at[dst_v.at[j]], add=True)
        return carry

    lax.fori_loop(0, NCHUNK, body, 0)
    plsc.subcore_barrier()
    pltpu.sync_copy(deg.at[pl.ds(row0, ROWS_PER_TILE)],
                    out_hbm.at[c].at[pl.ds(row0, ROWS_PER_TILE)])


@functools.partial(
    pl.kernel,
    out_type=jax.ShapeDtypeStruct((NC, N_PAD, D), jnp.float32),
    mesh=_sc_mesh,
    scratch_types=[
        pltpu.VMEM((NCHUNK // 2, CHUNK), jnp.int32),
        pltpu.VMEM((NCHUNK // 2, CHUNK), jnp.int32),
        pltpu.VMEM((CHUNK, D), jnp.float32),
        pltpu.VMEM((CHUNK, D), jnp.float32),
        pltpu.VMEM_SHARED((N_PAD, D), jnp.float32),
        pltpu.SemaphoreType.DMA,
        pltpu.SemaphoreType.DMA,
    ],
)
def _sc_agg(src_hbm, dst_hbm, hs_hbm, out_hbm, src_v, dst_v, buf_a, buf_b,
            acc, sem_a, sem_b):
    c = lax.axis_index("c")
    s = lax.axis_index("s")
    wid = c * NS + s
    row0 = s * ROWS_PER_TILE
    pltpu.sync_copy(hs_hbm.at[pl.ds(row0, ROWS_PER_TILE)],
                    acc.at[pl.ds(row0, ROWS_PER_TILE)])
    plsc.subcore_barrier()

    def _start(j, buf, sem):
        pltpu.async_copy(hs_hbm.at[src_v.at[j]], buf, sem)

    def _finish(j, buf, sem):
        pltpu.make_async_copy(hs_hbm.at[src_v.at[j]], buf, sem).wait()
        pltpu.sync_copy(buf, acc.at[dst_v.at[j]], add=True)

    wch = NCHUNK // 2
    npair = wch // 2
    for w in range(2):
        pltpu.sync_copy(src_hbm.at[wid].at[pl.ds(w * wch, wch)], src_v)
        pltpu.sync_copy(dst_hbm.at[wid].at[pl.ds(w * wch, wch)], dst_v)
        def body(j, carry):
            pltpu.async_copy(hs_hbm.at[src_v.at[j]], buf_a, sem_a).wait()
            pltpu.sync_copy(buf_a, acc.at[dst_v.at[j]], add=True)
            return carry

        lax.fori_loop(0, wch, body, 0)

    plsc.subcore_barrier()
    pltpu.sync_copy(acc.at[pl.ds(row0, ROWS_PER_TILE)],
                    out_hbm.at[c].at[pl.ds(row0, ROWS_PER_TILE)])


def _dinv(d0_ref, d1_ref):
    deg = d0_ref[:, :1] + d1_ref[:, :1] - 1.0
    return lax.rsqrt(deg)


def _pre_body(x_ref, d0_ref, d1_ref, w_ref, o_ref):
    xs = x_ref[...] * _dinv(d0_ref, d1_ref)
    o_ref[...] = jnp.dot(xs, w_ref[...], preferred_element_type=jnp.float32,
                         precision=lax.Precision.HIGHEST)


def _mid_body(p0_ref, p1_ref, hs_ref, d0_ref, d1_ref, b_ref, w_ref, o_ref):
    dinv = _dinv(d0_ref, d1_ref)
    agg = p0_ref[...] + p1_ref[...] - hs_ref[...]
    h = jnp.maximum(agg * dinv + b_ref[...], 0.0)
    o_ref[...] = jnp.dot(h * dinv, w_ref[...],
                         preferred_element_type=jnp.float32,
                         precision=lax.Precision.HIGHEST)


def _fin_body(p0_ref, p1_ref, hs_ref, d0_ref, d1_ref, b_ref, w_ref, bf_ref,
              o_ref):
    dinv = _dinv(d0_ref, d1_ref)
    agg = p0_ref[...] + p1_ref[...] - hs_ref[...]
    h = jnp.maximum(agg * dinv + b_ref[...], 0.0)
    o_ref[...] = jnp.dot(h, w_ref[...], preferred_element_type=jnp.float32,
                         precision=lax.Precision.HIGHEST) + bf_ref[...]


def _row_spec(rows, cols):
    return pl.BlockSpec((rows, cols), lambda i: (i, 0))


def _full_spec(r, c):
    return pl.BlockSpec((r, c), lambda i: (0, 0))


_tc_pre = pl.pallas_call(
    _pre_body,
    grid=(N_PAD // RBLK,),
    in_specs=[_row_spec(RBLK, D), _row_spec(RBLK, 16), _row_spec(RBLK, 16),
              _full_spec(D, D)],
    out_specs=_row_spec(RBLK, D),
    out_shape=jax.ShapeDtypeStruct((N_PAD, D), jnp.float32),
)

_tc_mid = pl.pallas_call(
    _mid_body,
    grid=(N_PAD // RBLK,),
    in_specs=[_row_spec(RBLK, D), _row_spec(RBLK, D), _row_spec(RBLK, D),
              _row_spec(RBLK, 16), _row_spec(RBLK, 16), _full_spec(1, D),
              _full_spec(D, D)],
    out_specs=_row_spec(RBLK, D),
    out_shape=jax.ShapeDtypeStruct((N_PAD, D), jnp.float32),
)

_FBLK = 1000

_tc_fin = pl.pallas_call(
    _fin_body,
    grid=(N_NODES // _FBLK,),
    in_specs=[_row_spec(_FBLK, D), _row_spec(_FBLK, D), _row_spec(_FBLK, D),
              _row_spec(_FBLK, 16), _row_spec(_FBLK, 16), _full_spec(1, D),
              _full_spec(D, 16), _full_spec(1, 16)],
    out_specs=_row_spec(_FBLK, 16),
    out_shape=jax.ShapeDtypeStruct((N_NODES, 16), jnp.float32),
)


def kernel(x, edge_index, W1, b1, W2, b2, Wfc, bfc):
    epad = jnp.full((E_PAD - N_EDGES,), N_PAD - 1, jnp.int32)
    src = jnp.concatenate([edge_index[0].astype(jnp.int32), epad])
    dst = jnp.concatenate([edge_index[1].astype(jnp.int32), epad])
    src = src.reshape(NW, NCHUNK, CHUNK)
    dst = dst.reshape(NW, NCHUNK, CHUNK)
    x = jnp.concatenate([x, jnp.zeros((N_PAD - N_NODES, D), x.dtype)])
    ones = jnp.ones((ROWS_PER_TILE, 16), jnp.float32)

    degp = _sc_deg(dst, ones)
    d0, d1 = degp[0], degp[1]
    b1r = b1.reshape(1, D)
    b2r = b2.reshape(1, D)
    bfr = bfc.reshape(1, 16)

    hs1 = _tc_pre(x, d0, d1, W1)
    p = _sc_agg(src, dst, hs1)
    hs2 = _tc_mid(p[0], p[1], hs1, d0, d1, b1r, W2)
    q = _sc_agg(src, dst, hs2)
    return _tc_fin(q[0], q[1], hs2, d0, d1, b2r, Wfc, bfr)


# speedup vs baseline: 2.0375x; 2.0375x over previous
import functools

import jax
import jax.numpy as jnp
from jax import lax
from jax.experimental import pallas as pl
from jax.experimental.pallas import tpu as pltpu
from jax.experimental.pallas import tpu_sc as plsc

N_NODES = 10000
N_PAD = 10240
N_EDGES = 320000
D = 128
NC = 2
NS = 16
NW = NC * NS
E_PER_W = N_EDGES // NW
CHUNK = 80
NCHUNK = E_PER_W // CHUNK
ROWS_PER_TILE = N_PAD // NS
RBLK = 1024

_sc_mesh = plsc.VectorSubcoreMesh(core_axis_name="c", subcore_axis_name="s")


@functools.partial(
    pl.kernel,
    out_type=jax.ShapeDtypeStruct((NC, N_PAD, 16), jnp.float32),
    mesh=_sc_mesh,
    scratch_types=[
        pltpu.VMEM((NCHUNK, CHUNK), jnp.int32),
        pltpu.VMEM((CHUNK, 16), jnp.float32),
        pltpu.VMEM_SHARED((N_PAD, 16), jnp.float32),
    ],
)
def _sc_deg(dst_hbm, ones_hbm, out_hbm, dst_v, ones_v, deg):
    c = lax.axis_index("c")
    s = lax.axis_index("s")
    wid = c * NS + s
    row0 = s * ROWS_PER_TILE
    pltpu.sync_copy(ones_hbm, deg.at[pl.ds(row0, ROWS_PER_TILE)])
    pltpu.sync_copy(ones_hbm.at[pl.ds(0, CHUNK)], ones_v)
    pltpu.sync_copy(dst_hbm.at[wid], dst_v)
    plsc.subcore_barrier()

    def body(j, carry):
        pltpu.sync_copy(ones_v, deg.at[dst_v.at[j]], add=True)
        return carry

    lax.fori_loop(0, NCHUNK, body, 0)
    plsc.subcore_barrier()
    pltpu.sync_copy(deg.at[pl.ds(row0, ROWS_PER_TILE)],
                    out_hbm.at[c].at[pl.ds(row0, ROWS_PER_TILE)])


@functools.partial(
    pl.kernel,
    out_type=jax.ShapeDtypeStruct((NC, N_PAD, D), jnp.float32),
    mesh=_sc_mesh,
    scratch_types=[
        pltpu.VMEM((NCHUNK, CHUNK), jnp.int32),
        pltpu.VMEM((NCHUNK, CHUNK), jnp.int32),
        pltpu.VMEM((CHUNK, D), jnp.float32),
        pltpu.VMEM_SHARED((N_PAD, D), jnp.float32),
    ],
)
def _sc_agg(src_hbm, dst_hbm, hs_hbm, out_hbm, src_v, dst_v, buf, acc):
    c = lax.axis_index("c")
    s = lax.axis_index("s")
    wid = c * NS + s
    row0 = s * ROWS_PER_TILE
    pltpu.sync_copy(hs_hbm.at[pl.ds(row0, ROWS_PER_TILE)],
                    acc.at[pl.ds(row0, ROWS_PER_TILE)])
    pltpu.sync_copy(src_hbm.at[wid], src_v)
    pltpu.sync_copy(dst_hbm.at[wid], dst_v)
    plsc.subcore_barrier()

    def body(j, carry):
        pltpu.sync_copy(hs_hbm.at[src_v.at[j]], buf)
        pltpu.sync_copy(buf, acc.at[dst_v.at[j]], add=True)
        return carry

    lax.fori_loop(0, NCHUNK, body, 0)
    plsc.subcore_barrier()
    pltpu.sync_copy(acc.at[pl.ds(row0, ROWS_PER_TILE)],
                    out_hbm.at[c].at[pl.ds(row0, ROWS_PER_TILE)])


def _dinv(d0_ref, d1_ref):
    deg = d0_ref[:, :1] + d1_ref[:, :1] - 1.0
    return lax.rsqrt(deg)


def _pre_body(x_ref, d0_ref, d1_ref, w_ref, o_ref):
    xs = x_ref[...] * _dinv(d0_ref, d1_ref)
    o_ref[...] = jnp.dot(xs, w_ref[...], preferred_element_type=jnp.float32,
                         precision=lax.Precision.HIGHEST)


def _mid_body(p0_ref, p1_ref, hs_ref, d0_ref, d1_ref, b_ref, w_ref, o_ref):
    dinv = _dinv(d0_ref, d1_ref)
    agg = p0_ref[...] + p1_ref[...] - hs_ref[...]
    h = jnp.maximum(agg * dinv + b_ref[...], 0.0)
    o_ref[...] = jnp.dot(h * dinv, w_ref[...],
                         preferred_element_type=jnp.float32,
                         precision=lax.Precision.HIGHEST)


def _fin_body(p0_ref, p1_ref, hs_ref, d0_ref, d1_ref, b_ref, w_ref, bf_ref,
              o_ref):
    dinv = _dinv(d0_ref, d1_ref)
    agg = p0_ref[...] + p1_ref[...] - hs_ref[...]
    h = jnp.maximum(agg * dinv + b_ref[...], 0.0)
    o_ref[...] = jnp.dot(h, w_ref[...], preferred_element_type=jnp.float32,
                         precision=lax.Precision.HIGHEST) + bf_ref[...]


def _row_spec(rows, cols):
    return pl.BlockSpec((rows, cols), lambda i: (i, 0))


def _full_spec(r, c):
    return pl.BlockSpec((r, c), lambda i: (0, 0))


_tc_pre = pl.pallas_call(
    _pre_body,
    grid=(N_PAD // RBLK,),
    in_specs=[_row_spec(RBLK, D), _row_spec(RBLK, 16), _row_spec(RBLK, 16),
              _full_spec(D, D)],
    out_specs=_row_spec(RBLK, D),
    out_shape=jax.ShapeDtypeStruct((N_PAD, D), jnp.float32),
)

_tc_mid = pl.pallas_call(
    _mid_body,
    grid=(N_PAD // RBLK,),
    in_specs=[_row_spec(RBLK, D), _row_spec(RBLK, D), _row_spec(RBLK, D),
              _row_spec(RBLK, 16), _row_spec(RBLK, 16), _full_spec(1, D),
              _full_spec(D, D)],
    out_specs=_row_spec(RBLK, D),
    out_shape=jax.ShapeDtypeStruct((N_PAD, D), jnp.float32),
)

_FBLK = 1000

_tc_fin = pl.pallas_call(
    _fin_body,
    grid=(N_NODES // _FBLK,),
    in_specs=[_row_spec(_FBLK, D), _row_spec(_FBLK, D), _row_spec(_FBLK, D),
              _row_spec(_FBLK, 16), _row_spec(_FBLK, 16), _full_spec(1, D),
              _full_spec(D, 16), _full_spec(1, 16)],
    out_specs=_row_spec(_FBLK, 16),
    out_shape=jax.ShapeDtypeStruct((N_NODES, 16), jnp.float32),
)


def kernel(x, edge_index, W1, b1, W2, b2, Wfc, bfc):
    src = edge_index[0].astype(jnp.int32).reshape(NW, NCHUNK, CHUNK)
    dst = edge_index[1].astype(jnp.int32).reshape(NW, NCHUNK, CHUNK)
    x = jnp.concatenate([x, jnp.zeros((N_PAD - N_NODES, D), x.dtype)])
    ones = jnp.ones((ROWS_PER_TILE, 16), jnp.float32)

    degp = _sc_deg(dst, ones)
    d0, d1 = degp[0], degp[1]
    b1r = b1.reshape(1, D)
    b2r = b2.reshape(1, D)
    bfr = bfc.reshape(1, 16)

    hs1 = _tc_pre(x, d0, d1, W1)
    p = _sc_agg(src, dst, hs1)
    hs2 = _tc_mid(p[0], p[1], hs1, d0, d1, b1r, W2)
    q = _sc_agg(src, dst, hs2)
    return _tc_fin(q[0], q[1], hs2, d0, d1, b2r, Wfc, bfr)


# speedup vs baseline: 2.9075x; 1.4270x over previous
import functools

import jax
import jax.numpy as jnp
from jax import lax
from jax.experimental import pallas as pl
from jax.experimental.pallas import tpu as pltpu
from jax.experimental.pallas import tpu_sc as plsc

N_NODES = 10000
N_PAD = 10240
N_EDGES = 320000
E_PAD = 327680
D = 128
NC = 2
NS = 16
NW = NC * NS
E_PER_W = E_PAD // NW
CHUNK = 80
NCHUNK = E_PER_W // CHUNK
WCH = NCHUNK // 2
ROWS_PER_TILE = N_PAD // NS
RBLK = 1024

_sc_mesh = plsc.VectorSubcoreMesh(core_axis_name="c", subcore_axis_name="s")


@functools.partial(
    pl.kernel,
    out_type=jax.ShapeDtypeStruct((NC, N_PAD, 16), jnp.float32),
    mesh=_sc_mesh,
    scratch_types=[
        pltpu.VMEM((2, WCH, CHUNK), jnp.int32),
        pltpu.VMEM((CHUNK, 16), jnp.float32),
        pltpu.VMEM_SHARED((N_PAD, 16), jnp.float32),
    ],
)
def _sc_deg(dst_hbm, ones_hbm, out_hbm, dst_v, ones_v, deg):
    c = lax.axis_index("c")
    s = lax.axis_index("s")
    wid = c * NS + s
    row0 = s * ROWS_PER_TILE
    pltpu.sync_copy(ones_hbm, deg.at[pl.ds(row0, ROWS_PER_TILE)])
    pltpu.sync_copy(ones_hbm.at[pl.ds(0, CHUNK)], ones_v)
    pltpu.sync_copy(dst_hbm.at[wid], dst_v)
    plsc.subcore_barrier()

    for w in range(2):
        def body(j, carry, w=w):
            pltpu.sync_copy(ones_v, deg.at[dst_v.at[w].at[j]], add=True)
            return carry

        lax.fori_loop(0, WCH, body, 0)
    plsc.subcore_barrier()
    pltpu.sync_copy(deg.at[pl.ds(row0, ROWS_PER_TILE)],
                    out_hbm.at[c].at[pl.ds(row0, ROWS_PER_TILE)])


@functools.partial(
    pl.kernel,
    out_type=jax.ShapeDtypeStruct((NC, N_PAD, D), jnp.float32),
    mesh=_sc_mesh,
    scratch_types=[
        pltpu.VMEM((WCH, CHUNK), jnp.int32),
        pltpu.VMEM((WCH, CHUNK), jnp.int32),
        pltpu.VMEM((CHUNK, D), jnp.float32),
        pltpu.VMEM((CHUNK, D), jnp.float32),
        pltpu.VMEM_SHARED((N_PAD, D), jnp.float32),
        pltpu.SemaphoreType.DMA,
        pltpu.SemaphoreType.DMA,
    ],
)
def _sc_agg(src_hbm, dst_hbm, hs_hbm, out_hbm, src_v, dst_v, buf_a, buf_b,
            acc, sem_a, sem_b):
    c = lax.axis_index("c")
    s = lax.axis_index("s")
    wid = c * NS + s
    row0 = s * ROWS_PER_TILE
    pltpu.sync_copy(hs_hbm.at[pl.ds(row0, ROWS_PER_TILE)],
                    acc.at[pl.ds(row0, ROWS_PER_TILE)])
    plsc.subcore_barrier()

    def _start(j, buf, sem):
        pltpu.async_copy(hs_hbm.at[src_v.at[j]], buf, sem)

    def _wait(j, buf, sem):
        pltpu.make_async_copy(hs_hbm.at[src_v.at[j]], buf, sem).wait()

    def _scat(j, buf):
        pltpu.sync_copy(buf, acc.at[dst_v.at[j]], add=True)

    for w in range(2):
        pltpu.sync_copy(src_hbm.at[wid].at[w], src_v)
        pltpu.sync_copy(dst_hbm.at[wid].at[w], dst_v)
        _start(0, buf_a, sem_a)

        def body(i, carry):
            j = 2 * i
            _start(j + 1, buf_b, sem_b)
            _wait(j, buf_a, sem_a)
            _scat(j, buf_a)

            @pl.when(j + 2 < WCH)
            def _():
                _start(j + 2, buf_a, sem_a)

            _wait(j + 1, buf_b, sem_b)
            _scat(j + 1, buf_b)
            return carry

        lax.fori_loop(0, WCH // 2, body, 0)

    plsc.subcore_barrier()
    pltpu.sync_copy(acc.at[pl.ds(row0, ROWS_PER_TILE)],
                    out_hbm.at[c].at[pl.ds(row0, ROWS_PER_TILE)])


def _dinv(d0_ref, d1_ref):
    deg = d0_ref[:, :1] + d1_ref[:, :1] - 1.0
    return lax.rsqrt(deg)


def _pre_body(x_ref, d0_ref, d1_ref, w_ref, o_ref):
    xs = x_ref[...] * _dinv(d0_ref, d1_ref)
    o_ref[...] = jnp.dot(xs, w_ref[...], preferred_element_type=jnp.float32,
                         precision=lax.Precision.HIGHEST)


def _mid_body(p0_ref, p1_ref, hs_ref, d0_ref, d1_ref, b_ref, w_ref, o_ref):
    dinv = _dinv(d0_ref, d1_ref)
    agg = p0_ref[...] + p1_ref[...] - hs_ref[...]
    h = jnp.maximum(agg * dinv + b_ref[...], 0.0)
    o_ref[...] = jnp.dot(h * dinv, w_ref[...],
                         preferred_element_type=jnp.float32,
                         precision=lax.Precision.HIGHEST)


def _fin_body(p0_ref, p1_ref, hs_ref, d0_ref, d1_ref, b_ref, w_ref, bf_ref,
              o_ref):
    dinv = _dinv(d0_ref, d1_ref)
    agg = p0_ref[...] + p1_ref[...] - hs_ref[...]
    h = jnp.maximum(agg * dinv + b_ref[...], 0.0)
    o_ref[...] = jnp.dot(h, w_ref[...], preferred_element_type=jnp.float32,
                         precision=lax.Precision.HIGHEST) + bf_ref[...]


def _row_spec(rows, cols):
    return pl.BlockSpec((rows, cols), lambda i: (i, 0))


def _full_spec(r, c):
    return pl.BlockSpec((r, c), lambda i: (0, 0))


_tc_pre = pl.pallas_call(
    _pre_body,
    grid=(N_PAD // RBLK,),
    in_specs=[_row_spec(RBLK, D), _row_spec(RBLK, 16), _row_spec(RBLK, 16),
              _full_spec(D, D)],
    out_specs=_row_spec(RBLK, D),
    out_shape=jax.ShapeDtypeStruct((N_PAD, D), jnp.float32),
)

_tc_mid = pl.pallas_call(
    _mid_body,
    grid=(N_PAD // RBLK,),
    in_specs=[_row_spec(RBLK, D), _row_spec(RBLK, D), _row_spec(RBLK, D),
              _row_spec(RBLK, 16), _row_spec(RBLK, 16), _full_spec(1, D),
              _full_spec(D, D)],
    out_specs=_row_spec(RBLK, D),
    out_shape=jax.ShapeDtypeStruct((N_PAD, D), jnp.float32),
)

_FBLK = 1000

_tc_fin = pl.pallas_call(
    _fin_body,
    grid=(N_NODES // _FBLK,),
    in_specs=[_row_spec(_FBLK, D), _row_spec(_FBLK, D), _row_spec(_FBLK, D),
              _row_spec(_FBLK, 16), _row_spec(_FBLK, 16), _full_spec(1, D),
              _full_spec(D, 16), _full_spec(1, 16)],
    out_specs=_row_spec(_FBLK, 16),
    out_shape=jax.ShapeDtypeStruct((N_NODES, 16), jnp.float32),
)


def kernel(x, edge_index, W1, b1, W2, b2, Wfc, bfc):
    pad_ix = N_NODES + (jnp.arange(E_PAD - N_EDGES, dtype=jnp.int32)
                        % (N_PAD - N_NODES))
    src = jnp.concatenate([edge_index[0].astype(jnp.int32), pad_ix])
    dst = jnp.concatenate([edge_index[1].astype(jnp.int32), pad_ix])
    src = src.reshape(NW, 2, WCH, CHUNK)
    dst = dst.reshape(NW, 2, WCH, CHUNK)
    x = jnp.concatenate([x, jnp.zeros((N_PAD - N_NODES, D), x.dtype)])
    ones = jnp.ones((ROWS_PER_TILE, 16), jnp.float32)

    degp = _sc_deg(dst, ones)
    d0, d1 = degp[0], degp[1]
    b1r = b1.reshape(1, D)
    b2r = b2.reshape(1, D)
    bfr = bfc.reshape(1, 16)

    hs1 = _tc_pre(x, d0, d1, W1)
    p = _sc_agg(src, dst, hs1)
    hs2 = _tc_mid(p[0], p[1], hs1, d0, d1, b1r, W2)
    q = _sc_agg(src, dst, hs2)
    return _tc_fin(q[0], q[1], hs2, d0, d1, b2r, Wfc, bfr)


# speedup vs baseline: 2.9519x; 1.0153x over previous
import functools

import jax
import jax.numpy as jnp
from jax import lax
from jax.experimental import pallas as pl
from jax.experimental.pallas import tpu as pltpu
from jax.experimental.pallas import tpu_sc as plsc

N_NODES = 10000
N_PAD = 10240
N_EDGES = 320000
E_PAD = 327680
D = 128
NC = 2
NS = 16
NW = NC * NS
E_PER_W = E_PAD // NW
CHUNK = 80
NCHUNK = E_PER_W // CHUNK
WCH = NCHUNK // 2
DCHUNK = 128
DCHUNKS = E_PER_W // DCHUNK
ROWS_PER_TILE = N_PAD // NS
RBLK = 1024

_sc_mesh = plsc.VectorSubcoreMesh(core_axis_name="c", subcore_axis_name="s")


@functools.partial(
    pl.kernel,
    out_type=jax.ShapeDtypeStruct((NC, N_PAD, 16), jnp.float32),
    mesh=_sc_mesh,
    scratch_types=[
        pltpu.VMEM((DCHUNKS, DCHUNK), jnp.int32),
        pltpu.VMEM((DCHUNK, 16), jnp.float32),
        pltpu.VMEM_SHARED((N_PAD, 16), jnp.float32),
    ],
)
def _sc_deg(dst_hbm, ones_hbm, out_hbm, dst_v, ones_v, deg):
    c = lax.axis_index("c")
    s = lax.axis_index("s")
    wid = c * NS + s
    row0 = s * ROWS_PER_TILE
    pltpu.sync_copy(ones_hbm, deg.at[pl.ds(row0, ROWS_PER_TILE)])
    pltpu.sync_copy(ones_hbm.at[pl.ds(0, DCHUNK)], ones_v)
    pltpu.sync_copy(dst_hbm.at[wid], dst_v)
    plsc.subcore_barrier()

    def body(j, carry):
        pltpu.sync_copy(ones_v, deg.at[dst_v.at[j]], add=True)
        return carry

    lax.fori_loop(0, DCHUNKS, body, 0)
    plsc.subcore_barrier()
    pltpu.sync_copy(deg.at[pl.ds(row0, ROWS_PER_TILE)],
                    out_hbm.at[c].at[pl.ds(row0, ROWS_PER_TILE)])


@functools.partial(
    pl.kernel,
    out_type=jax.ShapeDtypeStruct((NC, N_PAD, D), jnp.float32),
    mesh=_sc_mesh,
    scratch_types=[
        pltpu.VMEM((WCH, CHUNK), jnp.int32),
        pltpu.VMEM((WCH, CHUNK), jnp.int32),
        pltpu.VMEM((CHUNK, D), jnp.float32),
        pltpu.VMEM((CHUNK, D), jnp.float32),
        pltpu.VMEM_SHARED((N_PAD, D), jnp.float32),
        pltpu.SemaphoreType.DMA,
        pltpu.SemaphoreType.DMA,
    ],
)
def _sc_agg(src_hbm, dst_hbm, hs_hbm, out_hbm, src_v, dst_v, buf_a, buf_b,
            acc, sem_a, sem_b):
    c = lax.axis_index("c")
    s = lax.axis_index("s")
    wid = c * NS + s
    row0 = s * ROWS_PER_TILE
    pltpu.sync_copy(hs_hbm.at[pl.ds(row0, ROWS_PER_TILE)],
                    acc.at[pl.ds(row0, ROWS_PER_TILE)])
    plsc.subcore_barrier()

    def _start(j, buf, sem):
        pltpu.async_copy(hs_hbm.at[src_v.at[j]], buf, sem)

    def _wait(j, buf, sem):
        pltpu.make_async_copy(hs_hbm.at[src_v.at[j]], buf, sem).wait()

    def _scat(j, buf):
        pltpu.sync_copy(buf, acc.at[dst_v.at[j]], add=True)

    for w in range(2):
        pltpu.sync_copy(src_hbm.at[wid].at[w], src_v)
        pltpu.sync_copy(dst_hbm.at[wid].at[w], dst_v)
        _start(0, buf_a, sem_a)

        def body(i, carry):
            j = 2 * i
            _start(j + 1, buf_b, sem_b)
            _wait(j, buf_a, sem_a)
            _scat(j, buf_a)

            @pl.when(j + 2 < WCH)
            def _():
                _start(j + 2, buf_a, sem_a)

            _wait(j + 1, buf_b, sem_b)
            _scat(j + 1, buf_b)
            return carry

        lax.fori_loop(0, WCH // 2, body, 0)

    plsc.subcore_barrier()
    pltpu.sync_copy(acc.at[pl.ds(row0, ROWS_PER_TILE)],
                    out_hbm.at[c].at[pl.ds(row0, ROWS_PER_TILE)])


def _dinv(d0_ref, d1_ref):
    deg = d0_ref[:, :1] + d1_ref[:, :1] - 1.0
    return lax.rsqrt(deg)


def _mm_body(x_ref, w_ref, o_ref):
    o_ref[...] = jnp.dot(x_ref[...], w_ref[...],
                         preferred_element_type=jnp.float32,
                         precision=lax.Precision.HIGHEST)


def _scale_body(u_ref, d0_ref, d1_ref, o_ref):
    o_ref[...] = u_ref[...] * _dinv(d0_ref, d1_ref)


def _mid_body(p0_ref, p1_ref, hs_ref, d0_ref, d1_ref, b_ref, w_ref, o_ref):
    dinv = _dinv(d0_ref, d1_ref)
    agg = p0_ref[...] + p1_ref[...] - hs_ref[...]
    h = jnp.maximum(agg * dinv + b_ref[...], 0.0)
    o_ref[...] = jnp.dot(h * dinv, w_ref[...],
                         preferred_element_type=jnp.float32,
                         precision=lax.Precision.HIGHEST)


def _fin_body(p0_ref, p1_ref, hs_ref, d0_ref, d1_ref, b_ref, w_ref, bf_ref,
              o_ref):
    dinv = _dinv(d0_ref, d1_ref)
    agg = p0_ref[...] + p1_ref[...] - hs_ref[...]
    h = jnp.maximum(agg * dinv + b_ref[...], 0.0)
    o_ref[...] = jnp.dot(h, w_ref[...], preferred_element_type=jnp.float32,
                         precision=lax.Precision.HIGHEST) + bf_ref[...]


def _row_spec(rows, cols):
    return pl.BlockSpec((rows, cols), lambda i: (i, 0))


def _full_spec(r, c):
    return pl.BlockSpec((r, c), lambda i: (0, 0))


_tc_mm = pl.pallas_call(
    _mm_body,
    grid=(N_PAD // RBLK,),
    in_specs=[_row_spec(RBLK, D), _full_spec(D, D)],
    out_specs=_row_spec(RBLK, D),
    out_shape=jax.ShapeDtypeStruct((N_PAD, D), jnp.float32),
)

_tc_scale = pl.pallas_call(
    _scale_body,
    grid=(N_PAD // RBLK,),
    in_specs=[_row_spec(RBLK, D), _row_spec(RBLK, 16), _row_spec(RBLK, 16)],
    out_specs=_row_spec(RBLK, D),
    out_shape=jax.ShapeDtypeStruct((N_PAD, D), jnp.float32),
)

_tc_mid = pl.pallas_call(
    _mid_body,
    grid=(N_PAD // RBLK,),
    in_specs=[_row_spec(RBLK, D), _row_spec(RBLK, D), _row_spec(RBLK, D),
              _row_spec(RBLK, 16), _row_spec(RBLK, 16), _full_spec(1, D),
              _full_spec(D, D)],
    out_specs=_row_spec(RBLK, D),
    out_shape=jax.ShapeDtypeStruct((N_PAD, D), jnp.float32),
)

_FBLK = 1000

_tc_fin = pl.pallas_call(
    _fin_body,
    grid=(N_NODES // _FBLK,),
    in_specs=[_row_spec(_FBLK, D), _row_spec(_FBLK, D), _row_spec(_FBLK, D),
              _row_spec(_FBLK, 16), _row_spec(_FBLK, 16), _full_spec(1, D),
              _full_spec(D, 16), _full_spec(1, 16)],
    out_specs=_row_spec(_FBLK, 16),
    out_shape=jax.ShapeDtypeStruct((N_NODES, 16), jnp.float32),
)


def kernel(x, edge_index, W1, b1, W2, b2, Wfc, bfc):
    pad_ix = N_NODES + (jnp.arange(E_PAD - N_EDGES, dtype=jnp.int32)
                        % (N_PAD - N_NODES))
    src = jnp.concatenate([edge_index[0].astype(jnp.int32), pad_ix])
    dst = jnp.concatenate([edge_index[1].astype(jnp.int32), pad_ix])
    dstd = dst.reshape(NW, DCHUNKS, DCHUNK)
    src = src.reshape(NW, 2, WCH, CHUNK)
    dst = dst.reshape(NW, 2, WCH, CHUNK)
    x = jnp.concatenate([x, jnp.zeros((N_PAD - N_NODES, D), x.dtype)])
    ones = jnp.ones((ROWS_PER_TILE, 16), jnp.float32)

    degp = _sc_deg(dstd, ones)
    u = _tc_mm(x, W1)
    d0, d1 = degp[0], degp[1]
    b1r = b1.reshape(1, D)
    b2r = b2.reshape(1, D)
    bfr = bfc.reshape(1, 16)

    hs1 = _tc_scale(u, d0, d1)
    p = _sc_agg(src, dst, hs1)
    hs2 = _tc_mid(p[0], p[1], hs1, d0, d1, b1r, W2)
    q = _sc_agg(src, dst, hs2)
    return _tc_fin(q[0], q[1], hs2, d0, d1, b2r, Wfc, bfr)
